# baseline (speedup 1.0000x reference)
import functools

import jax
import jax.numpy as jnp
from jax import lax
from jax.experimental import pallas as pl
from jax.experimental.pallas import tpu as pltpu
from jax.experimental.pallas import tpu_sc as plsc

EMBED_D = 64
IDX_W = 128
K = 4


def _sc_gather(idx2d, table):
    nrows_idx = idx2d.shape[0]
    info = plsc.get_sparse_core_info()
    nc, ns = info.num_cores, info.num_subcores
    nw = nc * ns
    rows_per_w = nrows_idx // nw
    n_chunks = rows_per_w // K
    n_half = n_chunks // 2
    chunk = K * IDX_W
    b_total = nrows_idx * IDX_W

    mesh = plsc.VectorSubcoreMesh(core_axis_name="c", subcore_axis_name="s")

    @functools.partial(
        pl.kernel,
        mesh=mesh,
        out_type=jax.ShapeDtypeStruct((b_total, EMBED_D), jnp.float32),
        scratch_types=[
            pltpu.VMEM((2, K, IDX_W), jnp.int32),
            pltpu.VMEM((2, chunk, EMBED_D), jnp.float32),
            pltpu.SemaphoreType.DMA,
            pltpu.SemaphoreType.DMA,
        ],
        compiler_params=pltpu.CompilerParams(use_tc_tiling_on_sc=False),
    )
    def k(table_hbm, idx_hbm, out_hbm, idx_v, rows_v, sem0, sem1):
        wid = lax.axis_index("s") * nc + lax.axis_index("c")
        row0 = wid * rows_per_w
        sems = (sem0, sem1)

        def fire(g, slot):
            rbase = row0 + g * K
            pltpu.sync_copy(idx_hbm.at[pl.ds(rbase, K)], idx_v.at[slot])
            for j in range(K):
                pltpu.async_copy(
                    table_hbm.at[idx_v.at[slot].at[j]],
                    rows_v.at[slot].at[pl.ds(j * IDX_W, IDX_W)],
                    sems[slot],
                )

        def drain(slot):
            pltpu.make_async_copy(
                out_hbm.at[pl.ds(0, chunk)], rows_v.at[slot], sems[slot]
            ).wait()

        def write_out(g, slot):
            rbase = row0 + g * K
            pltpu.sync_copy(rows_v.at[slot], out_hbm.at[pl.ds(rbase * IDX_W, chunk)])

        fire(0, 0)
        fire(1, 1)

        def body(g2, carry):
            a = 2 * g2
            drain(0)
            write_out(a, 0)

            @pl.when(g2 < n_half - 1)
            def _():
                fire(a + 2, 0)

            drain(1)
            write_out(a + 1, 1)

            @pl.when(g2 < n_half - 1)
            def _():
                fire(a + 3, 1)

            return carry

        lax.fori_loop(0, n_half, body, 0)

    return k(table, idx2d)


def kernel(time_idx, table):
    b, s = time_idx.shape
    idx2d = time_idx.reshape((b * s) // IDX_W, IDX_W)
    out = _sc_gather(idx2d, table)
    return out.reshape(b, s, EMBED_D)
